# baseline (speedup 1.0000x reference)
import jax
import jax.numpy as jnp
from jax.experimental import pallas as pl


_BS = 512


def _add_kernel(x_ref, pos_ref, o_ref):
    o_ref[...] = x_ref[...] + pos_ref[...]


def kernel(x, pos_emb):
    batch, seq_len, emb = x.shape
    grid = (seq_len // _BS,)
    return pl.pallas_call(
        _add_kernel,
        grid=grid,
        in_specs=[
            pl.BlockSpec((batch, _BS, emb), lambda s: (0, s, 0)),
            pl.BlockSpec((_BS, emb), lambda s: (s, 0)),
        ],
        out_specs=pl.BlockSpec((batch, _BS, emb), lambda s: (0, s, 0)),
        out_shape=jax.ShapeDtypeStruct(x.shape, x.dtype),
    )(x, pos_emb)
